# Initial kernel scaffold; baseline (speedup 1.0000x reference)
import jax
import jax.numpy as jnp
from jax.experimental import pallas as pl


def kernel(node_h, edge_h, edge_index, edge_attr, W_e1, b_e1, W_e2, b_e2, W_n1, b_n1, W_n2, b_n2, ln_e_g, ln_e_b, ln_n_g, ln_n_b):
    raise NotImplementedError("write your pallas kernel here")


# speedup vs baseline: 2.4626x; 2.4626x over previous
import functools

import jax
import jax.numpy as jnp
from jax import lax
from jax.experimental import pallas as pl
from jax.experimental.pallas import tpu as pltpu
from jax.experimental.pallas import tpu_sc as plsc

N_NODES = 10000
N_EDGES = 320000
H = 128
EA = 16

_INFO = plsc.get_sparse_core_info()
NC = _INFO.num_cores
NS = _INFO.num_subcores
NW = NC * NS
EPW = N_EDGES // NW
CH = 80
NCH = EPW // CH
N_PAD = 10240
ROWS_PER_TILE = N_PAD // NS

_mesh = plsc.VectorSubcoreMesh(core_axis_name="c", subcore_axis_name="s")


@functools.partial(
    pl.kernel,
    out_type=(
        jax.ShapeDtypeStruct((N_EDGES, H), jnp.float32),
        jax.ShapeDtypeStruct((N_EDGES, H), jnp.float32),
    ),
    mesh=_mesh,
    scratch_types=[
        pltpu.VMEM((NCH, CH), jnp.int32),
        pltpu.VMEM((NCH, CH), jnp.int32),
        pltpu.VMEM((CH, H), jnp.float32),
        pltpu.VMEM((CH, H), jnp.float32),
        pltpu.SemaphoreType.DMA,
        pltpu.SemaphoreType.DMA,
    ],
)
def _sc_gather(node_hbm, src3_hbm, dst3_hbm, hs_hbm, hd_hbm,
               idxs_v, idxd_v, rows_s, rows_d, sem_s, sem_d):
    c = lax.axis_index("c")
    s = lax.axis_index("s")
    wid = s * NC + c
    base_ch = wid * NCH
    pltpu.sync_copy(src3_hbm.at[wid], idxs_v)
    pltpu.sync_copy(dst3_hbm.at[wid], idxd_v)

    def body(j, carry):
        ebase = (base_ch + j) * CH
        cp1 = pltpu.async_copy(node_hbm.at[idxs_v.at[j]], rows_s, sem_s)
        cp2 = pltpu.async_copy(node_hbm.at[idxd_v.at[j]], rows_d, sem_d)
        cp1.wait()
        cp2.wait()
        pltpu.sync_copy(rows_s, hs_hbm.at[pl.ds(ebase, CH)])
        pltpu.sync_copy(rows_d, hd_hbm.at[pl.ds(ebase, CH)])
        return carry

    lax.fori_loop(0, NCH, body, 0)


@functools.partial(
    pl.kernel,
    out_type=jax.ShapeDtypeStruct((NC, N_PAD, H), jnp.float32),
    mesh=_mesh,
    scratch_types=[
        pltpu.VMEM((NCH, CH), jnp.int32),
        pltpu.VMEM((CH, H), jnp.float32),
        pltpu.VMEM_SHARED((N_PAD, H), jnp.float32),
    ],
)
def _sc_scatter(ehn_hbm, dst3_hbm, zeros_hbm, out_hbm, idx_v, rows_v, agg_sh):
    c = lax.axis_index("c")
    s = lax.axis_index("s")
    wid = s * NC + c
    base_ch = wid * NCH

    row0 = s * ROWS_PER_TILE
    pltpu.sync_copy(zeros_hbm.at[pl.ds(row0, ROWS_PER_TILE)],
                    agg_sh.at[pl.ds(row0, ROWS_PER_TILE)])
    plsc.subcore_barrier()

    pltpu.sync_copy(dst3_hbm.at[wid], idx_v)

    def body(j, carry):
        ebase = (base_ch + j) * CH
        pltpu.sync_copy(ehn_hbm.at[pl.ds(ebase, CH)], rows_v)
        pltpu.sync_copy(rows_v, agg_sh.at[idx_v.at[j]], add=True)
        return carry

    lax.fori_loop(0, NCH, body, 0)
    plsc.subcore_barrier()

    pltpu.sync_copy(agg_sh.at[pl.ds(row0, ROWS_PER_TILE)],
                    out_hbm.at[c, pl.ds(row0, ROWS_PER_TILE)])


E_BLK = 512


def _edge_body(hs_ref, hd_ref, ea_ref, eh_ref, at_ref, bt_ref, ct_ref,
               w2t_ref, b1_ref, b2_ref, g_ref, bb_ref, out_ref):
    x = (jnp.dot(hs_ref[...], at_ref[...], preferred_element_type=jnp.float32)
         + jnp.dot(hd_ref[...], bt_ref[...], preferred_element_type=jnp.float32)
         + jnp.dot(ea_ref[...], ct_ref[...], preferred_element_type=jnp.float32)
         + b1_ref[...])
    h = jnp.maximum(x, 0.0)
    msg = jnp.dot(h, w2t_ref[...], preferred_element_type=jnp.float32) + b2_ref[...]
    y = eh_ref[...] + msg
    mu = jnp.mean(y, axis=-1, keepdims=True)
    var = jnp.mean((y - mu) ** 2, axis=-1, keepdims=True)
    out_ref[...] = (y - mu) / jnp.sqrt(var + 1e-5) * g_ref[...] + bb_ref[...]


def _edge_mlp(hs, hd, ea, eh, at, bt, ct, w2t, b1, b2, g, bb):
    grid = (N_EDGES // E_BLK,)
    blk = lambda r, cdim: pl.BlockSpec((E_BLK, cdim), lambda i: (i, 0)) if r else \
        pl.BlockSpec((None, cdim), lambda i: (0, 0))
    full = lambda shape: pl.BlockSpec(shape, lambda i: (0, 0))
    return pl.pallas_call(
        _edge_body,
        grid=grid,
        in_specs=[
            pl.BlockSpec((E_BLK, H), lambda i: (i, 0)),
            pl.BlockSpec((E_BLK, H), lambda i: (i, 0)),
            pl.BlockSpec((E_BLK, EA), lambda i: (i, 0)),
            pl.BlockSpec((E_BLK, H), lambda i: (i, 0)),
            full((H, 2 * H)),
            full((H, 2 * H)),
            full((EA, 2 * H)),
            full((2 * H, H)),
            full((1, 2 * H)),
            full((1, H)),
            full((1, H)),
            full((1, H)),
        ],
        out_specs=pl.BlockSpec((E_BLK, H), lambda i: (i, 0)),
        out_shape=jax.ShapeDtypeStruct((N_EDGES, H), jnp.float32),
        compiler_params=pltpu.CompilerParams(
            dimension_semantics=("arbitrary",)),
    )(hs, hd, ea, eh, at, bt, ct, w2t, b1, b2, g, bb)


N_BLK = 1000


def _node_body(nh_ref, a0_ref, a1_ref, dt_ref, et_ref, w2t_ref, b1_ref,
               b2_ref, g_ref, bb_ref, out_ref):
    agg = a0_ref[...] + a1_ref[...]
    x = (jnp.dot(nh_ref[...], dt_ref[...], preferred_element_type=jnp.float32)
         + jnp.dot(agg, et_ref[...], preferred_element_type=jnp.float32)
         + b1_ref[...])
    h = jnp.maximum(x, 0.0)
    upd = jnp.dot(h, w2t_ref[...], preferred_element_type=jnp.float32) + b2_ref[...]
    y = nh_ref[...] + upd
    mu = jnp.mean(y, axis=-1, keepdims=True)
    var = jnp.mean((y - mu) ** 2, axis=-1, keepdims=True)
    out_ref[...] = (y - mu) / jnp.sqrt(var + 1e-5) * g_ref[...] + bb_ref[...]


def _node_mlp(nh, a0, a1, dt, et, w2t, b1, b2, g, bb):
    grid = (N_NODES // N_BLK,)
    full = lambda shape: pl.BlockSpec(shape, lambda i: (0, 0))
    return pl.pallas_call(
        _node_body,
        grid=grid,
        in_specs=[
            pl.BlockSpec((N_BLK, H), lambda i: (i, 0)),
            pl.BlockSpec((N_BLK, H), lambda i: (i, 0)),
            pl.BlockSpec((N_BLK, H), lambda i: (i, 0)),
            full((H, 2 * H)),
            full((H, 2 * H)),
            full((2 * H, H)),
            full((1, 2 * H)),
            full((1, H)),
            full((1, H)),
            full((1, H)),
        ],
        out_specs=pl.BlockSpec((N_BLK, H), lambda i: (i, 0)),
        out_shape=jax.ShapeDtypeStruct((N_NODES, H), jnp.float32),
        compiler_params=pltpu.CompilerParams(
            dimension_semantics=("arbitrary",)),
    )(nh, a0, a1, dt, et, w2t, b1, b2, g, bb)


def kernel(node_h, edge_h, edge_index, edge_attr,
           W_e1, b_e1, W_e2, b_e2, W_n1, b_n1, W_n2, b_n2,
           ln_e_g, ln_e_b, ln_n_g, ln_n_b):
    ei = edge_index.astype(jnp.int32)
    src3 = ei[0].reshape(NW, NCH, CH)
    dst3 = ei[1].reshape(NW, NCH, CH)

    hs, hd = _sc_gather(node_h, src3, dst3)

    at = W_e1[:, :H].T
    bt = W_e1[:, H:2 * H].T
    ct = W_e1[:, 2 * H:].T
    w2t = W_e2.T
    edge_h_new = _edge_mlp(hs, hd, edge_attr, edge_h, at, bt, ct, w2t,
                           b_e1.reshape(1, -1), b_e2.reshape(1, -1),
                           ln_e_g.reshape(1, -1), ln_e_b.reshape(1, -1))

    zeros_pad = jnp.zeros((N_PAD, H), jnp.float32)
    parts = _sc_scatter(edge_h_new, dst3, zeros_pad)
    p0 = parts[0, :N_NODES]
    p1 = parts[1, :N_NODES]

    dt = W_n1[:, :H].T
    et = W_n1[:, H:].T
    wn2t = W_n2.T
    node_h_new = _node_mlp(node_h, p0, p1, dt, et, wn2t,
                           b_n1.reshape(1, -1), b_n2.reshape(1, -1),
                           ln_n_g.reshape(1, -1), ln_n_b.reshape(1, -1))
    return (node_h_new, edge_h_new)
